# baseline (speedup 1.0000x reference)
import functools

import jax
import jax.numpy as jnp
from jax import lax
from jax.experimental import pallas as pl
from jax.experimental.pallas import tpu as pltpu
from jax.experimental.pallas import tpu_sc as plsc

N = 10000
D = 128
HID = 128
HH = HID // 2
E = 320000
B = 100000
N_PAD = 10240

NC = 2
NS = 16
NW = NC * NS
L = 16

CH = 128
SUP = 8
SUPE = SUP * CH

EPT = ((E + NS * SUPE - 1) // (NS * SUPE)) * SUPE
E_PAD = EPT * NS
NSUP = EPT // SUPE
PPT = ((B + NW * CH - 1) // (NW * CH)) * CH
B_PAD = PPT * NW
ROWS_PT = N_PAD // NS

_mesh = plsc.VectorSubcoreMesh(core_axis_name="c", subcore_axis_name="s",
                               num_cores=NC, num_subcores=NS)


def _proj_body(x_ref, w_ref, o_ref):
    o_ref[0] = jnp.dot(x_ref[...], w_ref[...],
                       preferred_element_type=jnp.float32)


def _proj0(x, wperm):
    return pl.pallas_call(
        _proj_body,
        grid=(2, 10),
        in_specs=[pl.BlockSpec((1024, D), lambda j, i: (i, 0)),
                  pl.BlockSpec((D, HID), lambda j, i: (0, j))],
        out_specs=pl.BlockSpec((1, 1024, HID), lambda j, i: (j, i, 0)),
        out_shape=jax.ShapeDtypeStruct((NC, N_PAD, HID), jnp.float32),
    )(x, wperm)


def _fuse_body(p_ref, b_ref, w_ref, o_ref):
    h = jnp.concatenate([p_ref[0, :, :HH], p_ref[1, :, :HH]], axis=1)
    h = jnp.maximum(h + b_ref[...], 0.0)
    o_ref[0] = jnp.dot(h, w_ref[...], preferred_element_type=jnp.float32)


def _proj_fused(parts, b, wperm):
    return pl.pallas_call(
        _fuse_body,
        grid=(2, 10),
        in_specs=[pl.BlockSpec((2, 1024, HID), lambda j, i: (0, i, 0)),
                  pl.BlockSpec((1, HID), lambda j, i: (0, 0)),
                  pl.BlockSpec((HID, HID), lambda j, i: (0, j))],
        out_specs=pl.BlockSpec((1, 1024, HID), lambda j, i: (j, i, 0)),
        out_shape=jax.ShapeDtypeStruct((NC, N_PAD, HID), jnp.float32),
    )(parts, b, wperm)


def _final_body(p_ref, b_ref, wd_ref, h_ref, hw_ref):
    h = jnp.concatenate([p_ref[0, :, :HH], p_ref[1, :, :HH]], axis=1)
    h = jnp.maximum(h + b_ref[...], 0.0)
    h_ref[...] = h
    hw_ref[...] = h * wd_ref[...]


def _final(parts, b, wd_row):
    return pl.pallas_call(
        _final_body,
        grid=(10,),
        in_specs=[pl.BlockSpec((2, 1024, HID), lambda i: (0, i, 0)),
                  pl.BlockSpec((1, HID), lambda i: (0, 0)),
                  pl.BlockSpec((1, HID), lambda i: (0, 0))],
        out_specs=(pl.BlockSpec((1024, HID), lambda i: (i, 0)),
                   pl.BlockSpec((1024, HID), lambda i: (i, 0))),
        out_shape=(jax.ShapeDtypeStruct((N_PAD, HID), jnp.float32),
                   jax.ShapeDtypeStruct((N_PAD, HID), jnp.float32)),
    )(parts, b, wd_row)


def _compute_msg(gb_v, sv0_v, sv1_v, soff):
    def grp_body(gi, c):
        svec0 = sv0_v[pl.ds(soff + gi * L, L)]
        svec1 = sv1_v[pl.ds(soff + gi * L, L)]
        for t in range(L):
            i = gi * L + t
            a0 = svec0[t]
            a1 = svec1[t]
            for j in range(HH // L):
                g0 = gb_v[i, pl.ds(j * L, L)]
                g1 = gb_v[i, pl.ds(HH + j * L, L)]
                gb_v[i, pl.ds(j * L, L)] = g0 * a0 + g1 * a1
        return c

    lax.fori_loop(0, CH // L, grp_body, 0, unroll=False)


def _layer_body(mcat_hbm, src_hbm, dst_hbm, sv0_hbm, sv1_hbm, zero_hbm,
                out_hbm, src_v, dst_v, sv0_v, sv1_v, gb0_v, gb1_v, acc_sh,
                sg0, sg1, sc0, sc1):
    cid = lax.axis_index("c")
    sid = lax.axis_index("s")
    gbs = (gb0_v, gb1_v)
    sgs = (sg0, sg1)
    scs = (sc0, sc1)

    pltpu.sync_copy(zero_hbm.at[pl.ds(sid * ROWS_PT, ROWS_PT)],
                    acc_sh.at[pl.ds(sid * ROWS_PT, ROWS_PT)])
    plsc.subcore_barrier()

    def super_body(s, carry):
        row0 = sid * (EPT // CH) + s * SUP
        pltpu.sync_copy(src_hbm.at[cid, pl.ds(row0, SUP), :], src_v)
        pltpu.sync_copy(dst_hbm.at[pl.ds(row0, SUP), :], dst_v)
        eb = sid * EPT + s * SUPE
        pltpu.sync_copy(sv0_hbm.at[pl.ds(eb, SUPE)], sv0_v)
        pltpu.sync_copy(sv1_hbm.at[pl.ds(eb, SUPE)], sv1_v)

        for j in range(SUP):
            pltpu.async_copy(mcat_hbm.at[dst_v.at[j]], gb0_v, sg0).wait()
        return carry

    lax.fori_loop(0, NSUP, super_body, 0, unroll=False)
    plsc.subcore_barrier()
    pltpu.sync_copy(acc_sh.at[pl.ds(sid * ROWS_PT, ROWS_PT)],
                    out_hbm.at[cid, pl.ds(sid * ROWS_PT, ROWS_PT)])


@functools.partial(
    pl.kernel,
    out_type=jax.ShapeDtypeStruct((NC, N_PAD, HID), jnp.float32),
    mesh=_mesh,
    scratch_types=[
        pltpu.VMEM((SUP, CH), jnp.int32),
        pltpu.VMEM((SUP, CH), jnp.int32),
        pltpu.VMEM((SUPE,), jnp.float32),
        pltpu.VMEM((SUPE,), jnp.float32),
        pltpu.VMEM((CH, 2 * HID), jnp.float32),
        pltpu.VMEM((CH, HID), jnp.float32),
        pltpu.VMEM_SHARED((N_PAD, HID), jnp.float32),
        pltpu.SemaphoreType.DMA,
        pltpu.SemaphoreType.DMA,
        pltpu.SemaphoreType.DMA,
        pltpu.SemaphoreType.DMA,
    ],
)
def _sc_layer(*args):
    _layer_body(*args)


def _hsum(v):
    idx = lax.iota(jnp.int32, L)
    for sh in (8, 4, 2, 1):
        v = v + v.at[idx ^ sh].get(mode="promise_in_bounds")
    return v


def _dec_body(hw_hbm, h_hbm, r_hbm, c_hbm, out_hbm, ridx_v, cidx_v,
              u_v, v_v, res_v, sem):
    wid = lax.axis_index("c") * NS + lax.axis_index("s")
    base = wid * PPT
    lane = lax.iota(jnp.int32, 16)

    def chunk_body(k, carry):
        pb = base + k * CH
        pltpu.sync_copy(r_hbm.at[pl.ds(pb, CH)], ridx_v)
        pltpu.sync_copy(c_hbm.at[pl.ds(pb, CH)], cidx_v)
        pltpu.async_copy(hw_hbm.at[ridx_v], u_v, sem).wait()
        pltpu.async_copy(h_hbm.at[cidx_v], v_v, sem).wait()

        def grp_body(g, c2):
            def pair_body(t, resv):
                i = g * L + t
                acc = u_v[i, pl.ds(0, L)] * v_v[i, pl.ds(0, L)]
                for j in range(1, HID // L):
                    acc = acc + u_v[i, pl.ds(j * L, L)] * v_v[i, pl.ds(j * L, L)]
                tot = _hsum(acc)
                return jnp.where(lane == t, tot, resv)

            resv = lax.fori_loop(0, L, pair_body,
                                 jnp.zeros((L,), jnp.float32))
            res_v[pl.ds(g * L, L)] = resv
            return c2

        lax.fori_loop(0, CH // L, grp_body, 0)
        pltpu.sync_copy(res_v, out_hbm.at[pl.ds(pb, CH)])
        return carry

    lax.fori_loop(0, PPT // CH, chunk_body, 0)


@functools.partial(
    pl.kernel,
    out_type=jax.ShapeDtypeStruct((B_PAD,), jnp.float32),
    mesh=_mesh,
    scratch_types=[
        pltpu.VMEM((CH,), jnp.int32),
        pltpu.VMEM((CH,), jnp.int32),
        pltpu.VMEM((CH, HID), jnp.float32),
        pltpu.VMEM((CH, HID), jnp.float32),
        pltpu.VMEM((CH,), jnp.float32),
        pltpu.SemaphoreType.DMA,
    ],
)
def _sc_decoder(*args):
    _dec_body(*args)


def _pad1(a, n, dtype):
    return jnp.pad(a.astype(dtype), (0, n - a.shape[0]))


def _perm_w(W):
    return jnp.concatenate(
        [W[0][:, :HH], W[1][:, :HH], W[0][:, HH:], W[1][:, HH:]], axis=1)


@jax.jit
def kernel(inputs, edge_index, support_values, r_indices, c_indices,
           W1, b1, W2, b2, Wd, bd):
    src1 = _pad1(edge_index[0], E_PAD, jnp.int32).reshape(E_PAD // CH, CH)
    src = jnp.stack([src1, src1 + N_PAD])
    dst = _pad1(edge_index[1], E_PAD, jnp.int32).reshape(E_PAD // CH, CH)
    sv0 = _pad1(support_values[0], E_PAD, jnp.float32)
    sv1 = _pad1(support_values[1], E_PAD, jnp.float32)
    r_idx = _pad1(r_indices, B_PAD, jnp.int32)
    c_idx = _pad1(c_indices, B_PAD, jnp.int32)
    wperm1 = _perm_w(W1)
    wperm2 = _perm_w(W2)
    zeros = jnp.zeros((N_PAD, HID), jnp.float32)
    x_pad = jnp.pad(inputs, ((0, N_PAD - N), (0, 0)))

    mcat1 = _proj0(x_pad, wperm1)
    p1 = _sc_layer(mcat1.reshape(N_PAD, 2 * HID), src, dst, sv0, sv1, zeros)
    mcat2 = _proj_fused(p1, b1.reshape(1, HID), wperm2)
    p2 = _sc_layer(mcat2.reshape(N_PAD, 2 * HID), src, dst, sv0, sv1, zeros)
    h2, h2w = _final(p2, b2.reshape(1, HID), Wd.reshape(1, HID))
    logits = _sc_decoder(h2w, h2, r_idx, c_idx)
    return logits[:B] + bd[0]


# speedup vs baseline: 1.1304x; 1.1304x over previous
import functools

import jax
import jax.numpy as jnp
from jax import lax
from jax.experimental import pallas as pl
from jax.experimental.pallas import tpu as pltpu
from jax.experimental.pallas import tpu_sc as plsc

N = 10000
D = 128
HID = 128
HH = HID // 2
E = 320000
B = 100000
N_PAD = 10240

NC = 2
NS = 16
NW = NC * NS
L = 16

CH = 128
CHL = 64
SUP = 8
SUPE = SUP * CHL

EPT = ((E + NW * SUPE - 1) // (NW * SUPE)) * SUPE
E_PAD = EPT * NW
NSUP = EPT // SUPE
PPT = ((B + NW * CH - 1) // (NW * CH)) * CH
B_PAD = PPT * NW
ROWS_PT = N_PAD // NS

_mesh = plsc.VectorSubcoreMesh(core_axis_name="c", subcore_axis_name="s",
                               num_cores=NC, num_subcores=NS)


def _proj_body(x_ref, w_ref, o_ref):
    o_ref[...] = jnp.dot(x_ref[...], w_ref[...],
                         preferred_element_type=jnp.float32)


def _proj0(x, wcat):
    return pl.pallas_call(
        _proj_body,
        grid=(10,),
        in_specs=[pl.BlockSpec((1024, D), lambda i: (i, 0)),
                  pl.BlockSpec((D, 2 * HID), lambda i: (0, 0))],
        out_specs=pl.BlockSpec((1024, 2 * HID), lambda i: (i, 0)),
        out_shape=jax.ShapeDtypeStruct((N_PAD, 2 * HID), jnp.float32),
    )(x, wcat)


def _fuse_body(p_ref, b_ref, w_ref, o_ref):
    h = jnp.maximum(p_ref[0] + p_ref[1] + b_ref[...], 0.0)
    o_ref[...] = jnp.dot(h, w_ref[...], preferred_element_type=jnp.float32)


def _proj_fused(parts, b, wcat):
    return pl.pallas_call(
        _fuse_body,
        grid=(10,),
        in_specs=[pl.BlockSpec((2, 1024, HID), lambda i: (0, i, 0)),
                  pl.BlockSpec((1, HID), lambda i: (0, 0)),
                  pl.BlockSpec((HID, 2 * HID), lambda i: (0, 0))],
        out_specs=pl.BlockSpec((1024, 2 * HID), lambda i: (i, 0)),
        out_shape=jax.ShapeDtypeStruct((N_PAD, 2 * HID), jnp.float32),
    )(parts, b, wcat)


def _final_body(p_ref, b_ref, wd_ref, h_ref, hw_ref):
    h = jnp.maximum(p_ref[0] + p_ref[1] + b_ref[...], 0.0)
    h_ref[...] = h
    hw_ref[...] = h * wd_ref[...]


def _final(parts, b, wd_row):
    return pl.pallas_call(
        _final_body,
        grid=(10,),
        in_specs=[pl.BlockSpec((2, 1024, HID), lambda i: (0, i, 0)),
                  pl.BlockSpec((1, HID), lambda i: (0, 0)),
                  pl.BlockSpec((1, HID), lambda i: (0, 0))],
        out_specs=(pl.BlockSpec((1024, HID), lambda i: (i, 0)),
                   pl.BlockSpec((1024, HID), lambda i: (i, 0))),
        out_shape=(jax.ShapeDtypeStruct((N_PAD, HID), jnp.float32),
                   jax.ShapeDtypeStruct((N_PAD, HID), jnp.float32)),
    )(parts, b, wd_row)


def _compute_msg(gb_v, msg_v, sv0_v, sv1_v, soff):
    def grp_body(gi, c):
        svec0 = sv0_v[pl.ds(soff + gi * L, L)]
        svec1 = sv1_v[pl.ds(soff + gi * L, L)]
        for t in range(L):
            i = gi * L + t
            a0 = svec0[t]
            a1 = svec1[t]
            for j in range(HID // L):
                g0 = gb_v[i, pl.ds(j * L, L)]
                g1 = gb_v[i, pl.ds(HID + j * L, L)]
                msg_v[i, pl.ds(j * L, L)] = g0 * a0 + g1 * a1
        return c

    lax.fori_loop(0, CHL // L, grp_body, 0, unroll=False)


def _layer_body(mcat_hbm, src_hbm, dst_hbm, sv0_hbm, sv1_hbm, zero_hbm,
                out_hbm, src_v, dst_v, sv0_v, sv1_v, gb0_v, gb1_v, msg_v,
                acc_sh, sg0, sg1):
    cid = lax.axis_index("c")
    sid = lax.axis_index("s")
    gbs = (gb0_v, gb1_v)
    sgs = (sg0, sg1)

    pltpu.sync_copy(zero_hbm.at[pl.ds(sid * ROWS_PT, ROWS_PT)],
                    acc_sh.at[pl.ds(sid * ROWS_PT, ROWS_PT)])
    plsc.subcore_barrier()

    def super_body(s, carry):
        row0 = (cid * NS + sid) * (EPT // CHL) + s * SUP
        pltpu.sync_copy(src_hbm.at[pl.ds(row0, SUP), :], src_v)
        pltpu.sync_copy(dst_hbm.at[pl.ds(row0, SUP), :], dst_v)
        eb = (cid * NS + sid) * EPT + s * SUPE
        pltpu.sync_copy(sv0_hbm.at[pl.ds(eb, SUPE)], sv0_v)
        pltpu.sync_copy(sv1_hbm.at[pl.ds(eb, SUPE)], sv1_v)

        def gather(j):
            return pltpu.make_async_copy(mcat_hbm.at[src_v.at[j]],
                                         gbs[j % 2], sgs[j % 2])

        gather(0).start()
        for j in range(SUP):
            if j + 1 < SUP:
                gather(j + 1).start()
            gather(j).wait()
            _compute_msg(gbs[j % 2], msg_v, sv0_v, sv1_v, j * CHL)
            pltpu.sync_copy(msg_v, acc_sh.at[dst_v.at[j]], add=True)
        return carry

    lax.fori_loop(0, NSUP, super_body, 0, unroll=False)
    plsc.subcore_barrier()
    pltpu.sync_copy(acc_sh.at[pl.ds(sid * ROWS_PT, ROWS_PT)],
                    out_hbm.at[cid, pl.ds(sid * ROWS_PT, ROWS_PT)])


@functools.partial(
    pl.kernel,
    out_type=jax.ShapeDtypeStruct((NC, N_PAD, HID), jnp.float32),
    mesh=_mesh,
    scratch_types=[
        pltpu.VMEM((SUP, CHL), jnp.int32),
        pltpu.VMEM((SUP, CHL), jnp.int32),
        pltpu.VMEM((SUPE,), jnp.float32),
        pltpu.VMEM((SUPE,), jnp.float32),
        pltpu.VMEM((CHL, 2 * HID), jnp.float32),
        pltpu.VMEM((CHL, 2 * HID), jnp.float32),
        pltpu.VMEM((CHL, HID), jnp.float32),
        pltpu.VMEM_SHARED((N_PAD, HID), jnp.float32),
        pltpu.SemaphoreType.DMA,
        pltpu.SemaphoreType.DMA,
    ],
)
def _sc_layer(*args):
    _layer_body(*args)


def _hsum(v):
    idx = lax.iota(jnp.int32, L)
    for sh in (8, 4, 2, 1):
        v = v + v.at[idx ^ sh].get(mode="promise_in_bounds")
    return v


def _dec_body(hw_hbm, h_hbm, r_hbm, c_hbm, out_hbm, ridx_v, cidx_v,
              u_v, v_v, res_v, sem, sem2):
    wid = lax.axis_index("c") * NS + lax.axis_index("s")
    base = wid * PPT
    lane = lax.iota(jnp.int32, 16)

    def chunk_body(k, carry):
        pb = base + k * CH
        pltpu.sync_copy(r_hbm.at[pl.ds(pb, CH)], ridx_v)
        pltpu.sync_copy(c_hbm.at[pl.ds(pb, CH)], cidx_v)
        du = pltpu.async_copy(hw_hbm.at[ridx_v], u_v, sem)
        dv = pltpu.async_copy(h_hbm.at[cidx_v], v_v, sem2)
        du.wait()
        dv.wait()

        def grp_body(g, c2):
            def pair_body(t, resv):
                i = g * L + t
                acc = u_v[i, pl.ds(0, L)] * v_v[i, pl.ds(0, L)]
                for j in range(1, HID // L):
                    acc = acc + u_v[i, pl.ds(j * L, L)] * v_v[i, pl.ds(j * L, L)]
                tot = _hsum(acc)
                return jnp.where(lane == t, tot, resv)

            resv = lax.fori_loop(0, L, pair_body,
                                 jnp.zeros((L,), jnp.float32))
            res_v[pl.ds(g * L, L)] = resv
            return c2

        lax.fori_loop(0, CH // L, grp_body, 0)
        pltpu.sync_copy(res_v, out_hbm.at[pl.ds(pb, CH)])
        return carry

    lax.fori_loop(0, PPT // CH, chunk_body, 0)


@functools.partial(
    pl.kernel,
    out_type=jax.ShapeDtypeStruct((B_PAD,), jnp.float32),
    mesh=_mesh,
    scratch_types=[
        pltpu.VMEM((CH,), jnp.int32),
        pltpu.VMEM((CH,), jnp.int32),
        pltpu.VMEM((CH, HID), jnp.float32),
        pltpu.VMEM((CH, HID), jnp.float32),
        pltpu.VMEM((CH,), jnp.float32),
        pltpu.SemaphoreType.DMA,
        pltpu.SemaphoreType.DMA,
    ],
)
def _sc_decoder(*args):
    _dec_body(*args)


def _pad1(a, n, dtype):
    return jnp.pad(a.astype(dtype), (0, n - a.shape[0]))


@jax.jit
def kernel(inputs, edge_index, support_values, r_indices, c_indices,
           W1, b1, W2, b2, Wd, bd):
    src = _pad1(edge_index[0], E_PAD, jnp.int32).reshape(E_PAD // CHL, CHL)
    dst = _pad1(edge_index[1], E_PAD, jnp.int32).reshape(E_PAD // CHL, CHL)
    sv0 = _pad1(support_values[0], E_PAD, jnp.float32)
    sv1 = _pad1(support_values[1], E_PAD, jnp.float32)
    r_idx = _pad1(r_indices, B_PAD, jnp.int32)
    c_idx = _pad1(c_indices, B_PAD, jnp.int32)
    wcat1 = jnp.concatenate([W1[0], W1[1]], axis=1)
    wcat2 = jnp.concatenate([W2[0], W2[1]], axis=1)
    zeros = jnp.zeros((N_PAD, HID), jnp.float32)
    x_pad = jnp.pad(inputs, ((0, N_PAD - N), (0, 0)))

    mcat1 = _proj0(x_pad, wcat1)
    p1 = _sc_layer(mcat1, src, dst, sv0, sv1, zeros)
    mcat2 = _proj_fused(p1, b1.reshape(1, HID), wcat2)
    p2 = _sc_layer(mcat2, src, dst, sv0, sv1, zeros)
    h2, h2w = _final(p2, b2.reshape(1, HID), Wd.reshape(1, HID))
    logits = _sc_decoder(h2w, h2, r_idx, c_idx)
    return logits[:B] + bd[0]
